# baseline (speedup 1.0000x reference)
import functools

import jax
import jax.numpy as jnp
from jax import lax
from jax.experimental import pallas as pl
from jax.experimental.pallas import tpu as pltpu
from jax.experimental.pallas import tpu_sc as plsc

D = 32
CHUNK = 128
LOOKAHEAD = 4
NBUF = 6
OB = 2


TB = 2


@functools.lru_cache(maxsize=None)
def _build_table_detranspose(vocab: int, nw: int):
    n_blk = vocab // CHUNK
    rem = vocab - n_blk * CHUNK
    n_iter = (n_blk + nw - 1) // nw
    ntr = D // 8
    mesh = plsc.VectorSubcoreMesh(core_axis_name="c", subcore_axis_name="s")

    @functools.partial(
        pl.kernel,
        mesh=mesh,
        out_type=jax.ShapeDtypeStruct((vocab * D,), jnp.float32),
        scratch_types=[
            *[pltpu.VMEM((ntr, 8, CHUNK), jnp.float32) for _ in range(TB)],
            *[pltpu.VMEM((CHUNK * D,), jnp.float32) for _ in range(TB)],
            pltpu.VMEM((max(rem, 1), D), jnp.float32),
            *[pltpu.SemaphoreType.DMA for _ in range(2 * TB)],
        ],
        compiler_params=pltpu.CompilerParams(
            use_tc_tiling_on_sc=True, needs_layout_passes=False
        ),
    )
    def detr_kernel(tblT_hbm, tail_hbm, out_hbm, *rest):
        tbufs = rest[:TB]
        obufs = rest[TB : 2 * TB]
        tailbuf = rest[2 * TB]
        g_sems = rest[2 * TB + 1 : 3 * TB + 1]
        w_sems = rest[3 * TB + 1 : 4 * TB + 1]
        wid = lax.axis_index("s") * 2 + lax.axis_index("c")
        iota32 = lax.iota(jnp.int32, 16) * D

        def blk(i):
            return wid + i * nw

        def tin_start(k, tb):
            for tr in range(ntr):
                pltpu.async_copy(
                    tblT_hbm.at[pl.ds(8 * tr, 8), pl.ds(k * CHUNK, CHUNK)],
                    tbufs[tb].at[tr],
                    g_sems[tb],
                )

        def tin_wait(k, tb):
            for tr in range(ntr):
                pltpu.make_async_copy(
                    tblT_hbm.at[pl.ds(8 * tr, 8), pl.ds(k * CHUNK, CHUNK)],
                    tbufs[tb].at[tr],
                    g_sems[tb],
                ).wait()

        def scatter(tb, n_rows=CHUNK):
            src = tbufs[tb]
            dst = obufs[tb]
            for m in range(n_rows // 16):
                for tr in range(ntr):
                    for s in range(8):
                        plsc.store_scatter(
                            dst,
                            [iota32 + (16 * m * D + 8 * tr + s)],
                            src[tr, s, 16 * m : 16 * (m + 1)],
                        )

        def wstart(k, tb):
            pltpu.async_copy(
                obufs[tb], out_hbm.at[pl.ds(k * CHUNK * D, CHUNK * D)],
                w_sems[tb],
            )

        def wdrain(k, tb):
            pltpu.make_async_copy(
                obufs[tb], out_hbm.at[pl.ds(k * CHUNK * D, CHUNK * D)],
                w_sems[tb],
            ).wait()

        for tb in range(TB):
            @pl.when(blk(tb) < n_blk)
            def _():
                tin_start(blk(tb), tb)

        def loop_body(g, carry):
            for tb in range(TB):
                i = g * TB + tb
                k = blk(i)

                @pl.when(k < n_blk)
                def _():
                    tin_wait(k, tb)
                    scatter(tb)

                    @pl.when(jnp.asarray(i >= TB))
                    def _():
                        wdrain(blk(i - TB), tb)

                    wstart(k, tb)

                    @pl.when(blk(i + TB) < n_blk)
                    def _():
                        tin_start(blk(i + TB), tb)

            return carry

        lax.fori_loop(0, (n_iter + TB - 1) // TB, loop_body, 0)
        n_valid = (n_blk - wid + nw - 1) // nw
        for tb in range(TB):
            i_cand = n_valid - 1
            i_last = jnp.where(i_cand % TB == tb, i_cand, i_cand - 1)

            @pl.when(i_last >= 0)
            def _():
                wdrain(blk(i_last), tb)

        if rem:
            @pl.when(wid == 0)
            def _():
                pltpu.sync_copy(tail_hbm, tailbuf)
                for r in range(rem):
                    for h in range(D // 16):
                        obufs[0][
                            pl.ds(r * D + 16 * h, 16)
                        ] = tailbuf[r, 16 * h : 16 * (h + 1)]
                pltpu.sync_copy(
                    obufs[0].at[pl.ds(0, rem * D)],
                    out_hbm.at[pl.ds(n_blk * CHUNK * D, rem * D)],
                )

    return detr_kernel


@functools.lru_cache(maxsize=None)
def _build_gather(n_fields: int, batch: int, vocab: int, nw: int):
    tcols = batch // CHUNK
    n_chunks = n_fields * tcols
    per_w = n_chunks // nw
    assert n_chunks % nw == 0 and LOOKAHEAD < NBUF
    assert (per_w - 2 * LOOKAHEAD) % NBUF == 0 and NBUF % OB == 0
    mesh = plsc.VectorSubcoreMesh(core_axis_name="c", subcore_axis_name="s")

    @functools.partial(
        pl.kernel,
        mesh=mesh,
        out_type=jax.ShapeDtypeStruct(
            (n_fields, D // 8, tcols, 8 * CHUNK), jnp.float32
        ),
        scratch_types=[
            pltpu.VMEM((per_w, CHUNK), jnp.int32),
            *[pltpu.VMEM((CHUNK, D), jnp.float32) for _ in range(NBUF)],
            *[pltpu.VMEM((D * CHUNK,), jnp.float32) for _ in range(OB)],
            *[pltpu.SemaphoreType.DMA for _ in range(NBUF + OB)],
        ],
        compiler_params=pltpu.CompilerParams(
            use_tc_tiling_on_sc=False, needs_layout_passes=False
        ),
    )
    def embed_kernel(idx_hbm, table_hbm, out_hbm, idx_v, *rest):
        rows = rest[:NBUF]
        obufs = rest[NBUF : NBUF + OB]
        g_sems = rest[NBUF + OB : 2 * NBUF + OB]
        w_sems = rest[2 * NBUF + OB : 2 * NBUF + 2 * OB]
        wid = lax.axis_index("s") * 2 + lax.axis_index("c")
        base = wid * per_w

        pltpu.sync_copy(idx_hbm.at[pl.ds(base, per_w)], idx_v)

        iota = lax.iota(jnp.int32, 16)

        def gstart(j, b):
            pltpu.async_copy(table_hbm.at[idx_v.at[j]], rows[b], g_sems[b])

        def gwait(j, b):
            pltpu.make_async_copy(
                table_hbm.at[idx_v.at[j]], rows[b], g_sems[b]
            ).wait()

        cvecs128 = [(iota + (16 * h)) * CHUNK for h in range(D // 16)]

        def transpose(b, ob):
            src = rows[b]
            dst = obufs[ob]

            def tbody(li, carry):
                for lu in range(16):
                    l = li * 16 + lu
                    for h in range(D // 16):
                        plsc.store_scatter(
                            dst, [cvecs128[h] + l],
                            src[l, 16 * h : 16 * (h + 1)],
                        )
                return carry

            lax.fori_loop(0, CHUNK // 16, tbody, 0)

        def out_tiles(j):
            g_id = base + j
            f = g_id // tcols
            tcol = g_id % tcols
            return [out_hbm.at[f, tr, tcol] for tr in range(D // 8)]

        def wstart(j, ob):
            for tr, dstt in enumerate(out_tiles(j)):
                pltpu.async_copy(
                    obufs[ob].at[pl.ds(tr * 8 * CHUNK, 8 * CHUNK)],
                    dstt, w_sems[ob],
                )

        def wdrain(j, ob):
            for tr, dstt in enumerate(out_tiles(j)):
                pltpu.make_async_copy(
                    obufs[ob].at[pl.ds(tr * 8 * CHUNK, 8 * CHUNK)],
                    dstt, w_sems[ob],
                ).wait()

        def step(j, b, ob, drain, launch_b):
            gwait(j, b)
            if drain:
                wdrain(j - OB, ob)
            transpose(b, ob)
            wstart(j, ob)
            if launch_b is not None:
                gstart(j + LOOKAHEAD, launch_b)

        for j in range(LOOKAHEAD):
            gstart(j, j % NBUF)
        for j in range(LOOKAHEAD):
            step(
                j, j % NBUF, j % OB,
                drain=j >= OB,
                launch_b=(j + LOOKAHEAD) % NBUF,
            )

        def body(g, carry):
            for u in range(NBUF):
                j = g * NBUF + LOOKAHEAD + u
                step(
                    j,
                    (LOOKAHEAD + u) % NBUF,
                    (LOOKAHEAD + u) % OB,
                    drain=True,
                    launch_b=(2 * LOOKAHEAD + u) % NBUF,
                )
            return carry

        lax.fori_loop(0, (per_w - 2 * LOOKAHEAD) // NBUF, body, 0)

        for j in range(per_w - LOOKAHEAD, per_w):
            step(j, j % NBUF, j % OB, drain=True, launch_b=None)
        for j in range(per_w - OB, per_w):
            wdrain(j, j % OB)

    return embed_kernel


def kernel(x, table):
    batch, n_fields = x.shape
    vocab = table.shape[0]
    info = plsc.get_sparse_core_info()
    nw = info.num_cores * info.num_subcores
    idx = x.T.astype(jnp.int32).reshape(n_fields * (batch // CHUNK), CHUNK)
    n_blk = vocab // CHUNK
    tail = table[n_blk * CHUNK :]
    tbl_lin = _build_table_detranspose(vocab, nw)(table.T, tail).reshape(
        vocab, D
    )
    k4 = _build_gather(n_fields, batch, vocab, nw)(idx, tbl_lin)
    k5 = k4.reshape(n_fields, D // 8, batch // CHUNK, 8, CHUNK)
    return k5.transpose(2, 4, 0, 1, 3).reshape(batch, n_fields, D)


# speedup vs baseline: 1.1781x; 1.1781x over previous
import functools

import jax
import jax.numpy as jnp
from jax import lax
from jax.experimental import pallas as pl
from jax.experimental.pallas import tpu as pltpu
from jax.experimental.pallas import tpu_sc as plsc

D = 32
CHUNK = 128
LOOKAHEAD = 4
NBUF = 6
OB = 2


@functools.lru_cache(maxsize=None)
def _build_gather(n_fields: int, batch: int, vocab: int, nw: int):
    tcols = batch // CHUNK
    n_chunks = n_fields * tcols
    per_w = n_chunks // nw
    assert n_chunks % nw == 0 and LOOKAHEAD < NBUF
    assert (per_w - 2 * LOOKAHEAD) % NBUF == 0 and NBUF % OB == 0
    mesh = plsc.VectorSubcoreMesh(core_axis_name="c", subcore_axis_name="s")

    @functools.partial(
        pl.kernel,
        mesh=mesh,
        out_type=jax.ShapeDtypeStruct(
            (n_fields, D // 8, tcols, 8 * CHUNK), jnp.float32
        ),
        scratch_types=[
            pltpu.VMEM((per_w, CHUNK), jnp.int32),
            *[pltpu.VMEM((CHUNK, D), jnp.float32) for _ in range(NBUF)],
            *[pltpu.VMEM((D * CHUNK,), jnp.float32) for _ in range(OB)],
            *[pltpu.SemaphoreType.DMA for _ in range(NBUF + OB)],
        ],
        compiler_params=pltpu.CompilerParams(
            use_tc_tiling_on_sc=False, needs_layout_passes=False
        ),
    )
    def embed_kernel(idx_hbm, table_hbm, out_hbm, idx_v, *rest):
        rows = rest[:NBUF]
        obufs = rest[NBUF : NBUF + OB]
        g_sems = rest[NBUF + OB : 2 * NBUF + OB]
        w_sems = rest[2 * NBUF + OB : 2 * NBUF + 2 * OB]
        wid = lax.axis_index("s") * 2 + lax.axis_index("c")
        base = wid * per_w

        pltpu.sync_copy(idx_hbm.at[pl.ds(base, per_w)], idx_v)

        iota = lax.iota(jnp.int32, 16)

        def gstart(j, b):
            pltpu.async_copy(table_hbm.at[idx_v.at[j]], rows[b], g_sems[b])

        def gwait(j, b):
            pltpu.make_async_copy(
                table_hbm.at[idx_v.at[j]], rows[b], g_sems[b]
            ).wait()

        cvecs128 = [(iota + (16 * h)) * CHUNK for h in range(D // 16)]

        def transpose(b, ob):
            src = rows[b]
            dst = obufs[ob]

            def tbody(li, carry):
                for lu in range(16):
                    l = li * 16 + lu
                    for h in range(D // 16):
                        plsc.store_scatter(
                            dst, [cvecs128[h] + l],
                            src[l, 16 * h : 16 * (h + 1)],
                        )
                return carry

            lax.fori_loop(0, CHUNK // 16, tbody, 0)

        def out_tiles(j):
            g_id = base + j
            f = g_id // tcols
            tcol = g_id % tcols
            return [out_hbm.at[f, tr, tcol] for tr in range(D // 8)]

        def wstart(j, ob):
            for tr, dstt in enumerate(out_tiles(j)):
                pltpu.async_copy(
                    obufs[ob].at[pl.ds(tr * 8 * CHUNK, 8 * CHUNK)],
                    dstt, w_sems[ob],
                )

        def wdrain(j, ob):
            for tr, dstt in enumerate(out_tiles(j)):
                pltpu.make_async_copy(
                    obufs[ob].at[pl.ds(tr * 8 * CHUNK, 8 * CHUNK)],
                    dstt, w_sems[ob],
                ).wait()

        def step(j, b, ob, drain, launch_b):
            gwait(j, b)
            if drain:
                wdrain(j - OB, ob)
            transpose(b, ob)
            wstart(j, ob)
            if launch_b is not None:
                gstart(j + LOOKAHEAD, launch_b)

        for j in range(LOOKAHEAD):
            gstart(j, j % NBUF)
        for j in range(LOOKAHEAD):
            step(
                j, j % NBUF, j % OB,
                drain=j >= OB,
                launch_b=(j + LOOKAHEAD) % NBUF,
            )

        def body(g, carry):
            for u in range(NBUF):
                j = g * NBUF + LOOKAHEAD + u
                step(
                    j,
                    (LOOKAHEAD + u) % NBUF,
                    (LOOKAHEAD + u) % OB,
                    drain=True,
                    launch_b=(2 * LOOKAHEAD + u) % NBUF,
                )
            return carry

        lax.fori_loop(0, (per_w - 2 * LOOKAHEAD) // NBUF, body, 0)

        for j in range(per_w - LOOKAHEAD, per_w):
            step(j, j % NBUF, j % OB, drain=True, launch_b=None)
        for j in range(per_w - OB, per_w):
            wdrain(j, j % OB)

    return embed_kernel


def kernel(x, table):
    batch, n_fields = x.shape
    vocab = table.shape[0]
    info = plsc.get_sparse_core_info()
    nw = info.num_cores * info.num_subcores
    idx = x.T.astype(jnp.int32).reshape(n_fields * (batch // CHUNK), CHUNK)
    k4 = _build_gather(n_fields, batch, vocab, nw)(idx, table)
    k5 = k4.reshape(n_fields, D // 8, batch // CHUNK, 8, CHUNK)
    return k5.transpose(2, 4, 0, 1, 3).reshape(batch, n_fields, D)
